# baseline (speedup 1.0000x reference)
import functools

import jax
import jax.numpy as jnp
from jax import lax
from jax.experimental import pallas as pl
from jax.experimental.pallas import tpu as pltpu
from jax.experimental.pallas import tpu_sc as plsc

N = 10000
E = 320000
F_IN = 128
H = 256
C = 64

NUM_SC = 2
NUM_TILES = 16
HALF = H // 2
E_PAD = 327680
EPT = E_PAD // NUM_TILES
BLK = 4096
N_BLKS = EPT // BLK
CHUNK = 128
CPB = BLK // CHUNK
N_CHUNKS = EPT // CHUNK
NPAIRS = N_CHUNKS // 2
NPAD = 10240
ROWS_PT = NPAD // NUM_TILES
MT = 1000


def _mm1_body(x_ref, w_ref, b_ref, o_ref):
    res = lax.dot_general(
        x_ref[...], w_ref[...], (((1,), (1,)), ((), ())),
        preferred_element_type=jnp.float32, precision=lax.Precision.HIGHEST)
    o_ref[...] = res[None] + b_ref[...]


def _mm1(x, w, b2):
    return pl.pallas_call(
        _mm1_body,
        grid=(NUM_SC, N // MT),
        in_specs=[
            pl.BlockSpec((MT, F_IN), lambda h, m: (m, 0)),
            pl.BlockSpec((HALF, F_IN), lambda h, m: (h, 0)),
            pl.BlockSpec((1, 1, HALF), lambda h, m: (h, 0, 0)),
        ],
        out_specs=pl.BlockSpec((1, MT, HALF), lambda h, m: (h, m, 0)),
        out_shape=jax.ShapeDtypeStruct((NUM_SC, N, HALF), jnp.float32),
    )(x, w, b2)


def _mm2_body(y_ref, w_ref, b_ref, o_ref):
    yv = jnp.maximum(y_ref[0], 0.0)
    part = lax.dot_general(
        yv, w_ref[...], (((1,), (1,)), ((), ())),
        preferred_element_type=jnp.float32, precision=lax.Precision.HIGHEST)

    @pl.when(pl.program_id(2) == 0)
    def _():
        o_ref[...] = part[None] + b_ref[...]

    @pl.when(pl.program_id(2) == 1)
    def _():
        o_ref[...] = o_ref[...] + part[None]


def _mm2(y, w, b2):
    return pl.pallas_call(
        _mm2_body,
        grid=(NUM_SC, N // MT, NUM_SC),
        in_specs=[
            pl.BlockSpec((1, MT, HALF), lambda h, m, c: (c, m, 0)),
            pl.BlockSpec((HALF, HALF), lambda h, m, c: (h, c)),
            pl.BlockSpec((1, 1, HALF), lambda h, m, c: (h, 0, 0)),
        ],
        out_specs=pl.BlockSpec((1, MT, HALF), lambda h, m, c: (h, m, 0)),
        out_shape=jax.ShapeDtypeStruct((NUM_SC, N, HALF), jnp.float32),
    )(y, w, b2)


def _mm3_body(y_ref, w_ref, b_ref, o_ref):
    yv = jnp.maximum(y_ref[0], 0.0)
    part = lax.dot_general(
        yv, w_ref[...], (((1,), (1,)), ((), ())),
        preferred_element_type=jnp.float32, precision=lax.Precision.HIGHEST)

    @pl.when(pl.program_id(1) == 0)
    def _():
        o_ref[...] = part + b_ref[...]

    @pl.when(pl.program_id(1) == 1)
    def _():
        z = jnp.maximum(o_ref[...] + part, 0.0)
        m = jnp.max(z, axis=-1, keepdims=True)
        s = z - m
        lse = jnp.log(jnp.sum(jnp.exp(s), axis=-1, keepdims=True))
        o_ref[...] = s - lse


def _mm3(y, w, b2):
    return pl.pallas_call(
        _mm3_body,
        grid=(N // MT, NUM_SC),
        in_specs=[
            pl.BlockSpec((1, MT, HALF), lambda m, c: (c, m, 0)),
            pl.BlockSpec((C, HALF), lambda m, c: (0, c)),
            pl.BlockSpec((1, C), lambda m, c: (0, 0)),
        ],
        out_specs=pl.BlockSpec((MT, C), lambda m, c: (m, 0)),
        out_shape=jax.ShapeDtypeStruct((N, C), jnp.float32),
    )(y, w, b2)


def _spmm_body(x_hbm, rows_hbm, cols_hbm, ev_hbm, zeros_hbm, out_hbm,
               acc, rows_v, cols_v, ev_v,
               gbuf0, gbuf1, ridx0, ridx1, cidx0, cidx1,
               gsem0, gsem1, ssem0, ssem1):
    tid = lax.axis_index("s")
    cid = lax.axis_index("c")
    pltpu.sync_copy(zeros_hbm, acc.at[pl.ds(tid * ROWS_PT, ROWS_PT)])
    plsc.subcore_barrier()

    x_h = x_hbm.at[cid]
    dn = lax.GatherDimensionNumbers(
        offset_dims=(), collapsed_slice_dims=(0,), start_index_map=(0,))

    def stage_block(bi):
        base = tid * EPT + bi * BLK
        pltpu.sync_copy(rows_hbm.at[pl.ds(base, BLK)], rows_v)
        pltpu.sync_copy(cols_hbm.at[pl.ds(base, BLK)], cols_v)
        pltpu.sync_copy(ev_hbm.at[pl.ds(base, BLK)], ev_v)

    def fire_gather(c, cidx, ridx, gbuf, gsem):
        off = (c % CPB) * CHUNK
        for j in range(CHUNK // 16):
            cidx[pl.ds(j * 16, 16)] = cols_v[pl.ds(off + j * 16, 16)]
            ridx[pl.ds(j * 16, 16)] = rows_v[pl.ds(off + j * 16, 16)]
        pltpu.async_copy(x_h.at[cidx], gbuf, gsem)

    def scale(c, gbuf):
        off = (c % CPB) * CHUNK

        @plsc.parallel_loop(0, CHUNK // 16, unroll=2)
        def _(g):
            evg = ev_v[pl.ds(off + g * 16, 16)]
            row0 = g * 16
            for l in range(16):
                evb = lax.gather(
                    evg, jnp.full((16, 1), l, jnp.int32), dn,
                    slice_sizes=(1,),
                    mode=lax.GatherScatterMode.PROMISE_IN_BOUNDS)
                for j in range(HALF // 16):
                    sl = pl.ds(j * 16, 16)
                    gbuf[row0 + l, sl] = gbuf[row0 + l, sl] * evb

    stage_block(0)
    fire_gather(0, cidx0, ridx0, gbuf0, gsem0)
    fire_gather(1, cidx1, ridx1, gbuf1, gsem1)

    def pair_body(p, _):
        c0 = 2 * p
        c1 = c0 + 1
        pltpu.make_async_copy(x_h.at[cidx0], gbuf0, gsem0).wait()
        scale(c0, gbuf0)
        s0 = pltpu.async_copy(gbuf0, acc.at[ridx0], ssem0, add=True)
        pltpu.make_async_copy(x_h.at[cidx1], gbuf1, gsem1).wait()
        scale(c1, gbuf1)
        s1 = pltpu.async_copy(gbuf1, acc.at[ridx1], ssem1, add=True)
        s0.wait()

        @pl.when(p < NPAIRS - 1)
        def _():
            @pl.when((c0 + 2) % CPB == 0)
            def _():
                stage_block((c0 + 2) // CPB)

            fire_gather(c0 + 2, cidx0, ridx0, gbuf0, gsem0)

        s1.wait()

        @pl.when(p < NPAIRS - 1)
        def _():
            fire_gather(c1 + 2, cidx1, ridx1, gbuf1, gsem1)

        return 0

    lax.fori_loop(0, NPAIRS, pair_body, 0)
    plsc.subcore_barrier()
    pltpu.sync_copy(acc.at[pl.ds(tid * ROWS_PT, ROWS_PT)],
                    out_hbm.at[cid, pl.ds(tid * ROWS_PT, ROWS_PT)])


_spmm_call = pl.kernel(
    _spmm_body,
    out_type=jax.ShapeDtypeStruct((NUM_SC, NPAD, HALF), jnp.float32),
    mesh=plsc.VectorSubcoreMesh(core_axis_name="c", subcore_axis_name="s"),
    scratch_types=[
        pltpu.MemorySpace.VMEM_SHARED((NPAD, HALF), jnp.float32),
        pltpu.VMEM((BLK,), jnp.int32),
        pltpu.VMEM((BLK,), jnp.int32),
        pltpu.VMEM((BLK,), jnp.float32),
    ] + [pltpu.VMEM((CHUNK, HALF), jnp.float32) for _ in range(2)]
    + [pltpu.VMEM((CHUNK,), jnp.int32) for _ in range(4)]
    + [pltpu.SemaphoreType.DMA for _ in range(4)],
)


def _spmm(x, rows, cols, ev, zeros):
    return _spmm_call(x, rows, cols, ev, zeros)


def kernel(edge_index, edge_values, embed, W1, b1, Wh, bh, Wt, bt):
    pad = E_PAD - E
    rows = jnp.concatenate([edge_index[0], jnp.zeros((pad,), jnp.int32)])
    cols = jnp.concatenate([edge_index[1], jnp.zeros((pad,), jnp.int32)])
    ev = jnp.concatenate([edge_values, jnp.zeros((pad,), jnp.float32)])
    zeros = jnp.zeros((ROWS_PT, HALF), jnp.float32)
    b1r = b1.reshape(NUM_SC, 1, HALF)
    bhr = bh.reshape(NUM_SC, 1, HALF)
    btr = bt.reshape(1, C)

    x1 = _mm1(embed, W1, b1r)
    y1 = _spmm(x1, rows, cols, ev, zeros)
    x2 = _mm2(y1, Wh, bhr)
    y2 = _spmm(x2, rows, cols, ev, zeros)
    return _mm3(y2, Wt, btr)


# speedup vs baseline: 2.5125x; 2.5125x over previous
import functools

import jax
import jax.numpy as jnp
from jax import lax
from jax.experimental import pallas as pl
from jax.experimental.pallas import tpu as pltpu
from jax.experimental.pallas import tpu_sc as plsc

N = 10000
E = 320000
F_IN = 128
H = 256
C = 64

NUM_SC = 2
NUM_TILES = 16
HALF = H // 2
E_PAD = 327680
EPT = E_PAD // NUM_TILES
BLK = 2560
N_BLKS = EPT // BLK
CHUNK = 80
CPB = BLK // CHUNK
N_CHUNKS = EPT // CHUNK
NPAIRS = N_CHUNKS // 2
NPAD = 10240
ROWS_PT = NPAD // NUM_TILES
MT = 1000


def _mm1_body(x_ref, w_ref, b_ref, o_ref):
    res = lax.dot_general(
        x_ref[...], w_ref[...], (((1,), (1,)), ((), ())),
        preferred_element_type=jnp.float32, precision=lax.Precision.HIGHEST)
    o_ref[...] = res[None] + b_ref[...]


def _mm1(x, w, b2):
    return pl.pallas_call(
        _mm1_body,
        grid=(NUM_SC, N // MT),
        in_specs=[
            pl.BlockSpec((MT, F_IN), lambda h, m: (m, 0)),
            pl.BlockSpec((HALF, F_IN), lambda h, m: (h, 0)),
            pl.BlockSpec((1, 1, HALF), lambda h, m: (h, 0, 0)),
        ],
        out_specs=pl.BlockSpec((1, MT, HALF), lambda h, m: (h, m, 0)),
        out_shape=jax.ShapeDtypeStruct((NUM_SC, N, HALF), jnp.float32),
    )(x, w, b2)


def _mm2_body(y_ref, w_ref, b_ref, o_ref):
    yv = jnp.maximum(y_ref[0], 0.0)
    part = lax.dot_general(
        yv, w_ref[...], (((1,), (1,)), ((), ())),
        preferred_element_type=jnp.float32, precision=lax.Precision.HIGHEST)

    @pl.when(pl.program_id(2) == 0)
    def _():
        o_ref[...] = part[None] + b_ref[...]

    @pl.when(pl.program_id(2) == 1)
    def _():
        o_ref[...] = o_ref[...] + part[None]


def _mm2(y, w, b2):
    return pl.pallas_call(
        _mm2_body,
        grid=(NUM_SC, N // MT, NUM_SC),
        in_specs=[
            pl.BlockSpec((1, MT, HALF), lambda h, m, c: (c, m, 0)),
            pl.BlockSpec((HALF, HALF), lambda h, m, c: (h, c)),
            pl.BlockSpec((1, 1, HALF), lambda h, m, c: (h, 0, 0)),
        ],
        out_specs=pl.BlockSpec((1, MT, HALF), lambda h, m, c: (h, m, 0)),
        out_shape=jax.ShapeDtypeStruct((NUM_SC, N, HALF), jnp.float32),
    )(y, w, b2)


def _mm3_body(y_ref, w_ref, b_ref, o_ref):
    yv = jnp.maximum(y_ref[0], 0.0)
    part = lax.dot_general(
        yv, w_ref[...], (((1,), (1,)), ((), ())),
        preferred_element_type=jnp.float32, precision=lax.Precision.HIGHEST)

    @pl.when(pl.program_id(1) == 0)
    def _():
        o_ref[...] = part + b_ref[...]

    @pl.when(pl.program_id(1) == 1)
    def _():
        z = jnp.maximum(o_ref[...] + part, 0.0)
        m = jnp.max(z, axis=-1, keepdims=True)
        s = z - m
        lse = jnp.log(jnp.sum(jnp.exp(s), axis=-1, keepdims=True))
        o_ref[...] = s - lse


def _mm3(y, w, b2):
    return pl.pallas_call(
        _mm3_body,
        grid=(N // MT, NUM_SC),
        in_specs=[
            pl.BlockSpec((1, MT, HALF), lambda m, c: (c, m, 0)),
            pl.BlockSpec((C, HALF), lambda m, c: (0, c)),
            pl.BlockSpec((1, C), lambda m, c: (0, 0)),
        ],
        out_specs=pl.BlockSpec((MT, C), lambda m, c: (m, 0)),
        out_shape=jax.ShapeDtypeStruct((N, C), jnp.float32),
    )(y, w, b2)


def _spmm_body(x_hbm, rows_hbm, cols_hbm, ev_hbm, zeros_hbm, out_hbm,
               acc, rows_v, cols_v, ev_v,
               gbuf0, gbuf1, sbuf0, sbuf1,
               sridx0, sridx1, cidx0, cidx1,
               gsem0, gsem1, ssem0, ssem1):
    tid = lax.axis_index("s")
    cid = lax.axis_index("c")
    pltpu.sync_copy(zeros_hbm, acc.at[pl.ds(tid * ROWS_PT, ROWS_PT)])
    plsc.subcore_barrier()

    x_h = x_hbm.at[cid]
    dn = lax.GatherDimensionNumbers(
        offset_dims=(), collapsed_slice_dims=(0,), start_index_map=(0,))

    def stage_block(bi):
        base = tid * EPT + bi * BLK
        pltpu.sync_copy(rows_hbm.at[pl.ds(base, BLK)], rows_v)
        pltpu.sync_copy(cols_hbm.at[pl.ds(base, BLK)], cols_v)
        pltpu.sync_copy(ev_hbm.at[pl.ds(base, BLK)], ev_v)

    def fire_gather(c, cidx, gbuf, gsem):
        off = (c % CPB) * CHUNK
        for j in range(CHUNK // 16):
            cidx[pl.ds(j * 16, 16)] = cols_v[pl.ds(off + j * 16, 16)]
        pltpu.async_copy(x_h.at[cidx], gbuf, gsem)

    def scale(c, gbuf, sbuf):
        off = (c % CPB) * CHUNK

        @plsc.parallel_loop(0, CHUNK // 16, unroll=2)
        def _(g):
            evg = ev_v[pl.ds(off + g * 16, 16)]
            row0 = g * 16
            for l in range(16):
                evb = lax.gather(
                    evg, jnp.full((16, 1), l, jnp.int32), dn,
                    slice_sizes=(1,),
                    mode=lax.GatherScatterMode.PROMISE_IN_BOUNDS)
                for j in range(HALF // 16):
                    sl = pl.ds(j * 16, 16)
                    sbuf[row0 + l, sl] = gbuf[row0 + l, sl] * evb

    def fire_scatter(c, sridx, sbuf, ssem):
        off = (c % CPB) * CHUNK
        for j in range(CHUNK // 16):
            sridx[pl.ds(j * 16, 16)] = rows_v[pl.ds(off + j * 16, 16)]
        pltpu.async_copy(sbuf, acc.at[sridx], ssem, add=True)

    stage_block(0)
    fire_gather(0, cidx0, gbuf0, gsem0)
    fire_gather(1, cidx1, gbuf1, gsem1)
    pltpu.make_async_copy(x_h.at[cidx0], gbuf0, gsem0).wait()
    scale(0, gbuf0, sbuf0)
    fire_scatter(0, sridx0, sbuf0, ssem0)
    fire_gather(2, cidx0, gbuf0, gsem0)
    pltpu.make_async_copy(x_h.at[cidx1], gbuf1, gsem1).wait()
    scale(1, gbuf1, sbuf1)
    fire_scatter(1, sridx1, sbuf1, ssem1)
    fire_gather(3, cidx1, gbuf1, gsem1)

    def pair_body(p, _):
        c0 = 2 * p
        c1 = c0 + 1
        nxt = p < NPAIRS - 1
        boundary = (c0 + 2) % CPB == 0
        pltpu.make_async_copy(x_h.at[cidx0], gbuf0, gsem0).wait()
        pltpu.make_async_copy(sbuf0, acc.at[sridx0], ssem0).wait()
        scale(c0, gbuf0, sbuf0)
        fire_scatter(c0, sridx0, sbuf0, ssem0)

        @pl.when(jnp.logical_and(nxt, jnp.logical_not(boundary)))
        def _():
            fire_gather(c0 + 2, cidx0, gbuf0, gsem0)

        pltpu.make_async_copy(x_h.at[cidx1], gbuf1, gsem1).wait()
        pltpu.make_async_copy(sbuf1, acc.at[sridx1], ssem1).wait()
        scale(c1, gbuf1, sbuf1)
        fire_scatter(c1, sridx1, sbuf1, ssem1)

        @pl.when(jnp.logical_and(nxt, boundary))
        def _():
            stage_block((c0 + 2) // CPB)
            fire_gather(c0 + 2, cidx0, gbuf0, gsem0)

        @pl.when(nxt)
        def _():
            fire_gather(c1 + 2, cidx1, gbuf1, gsem1)

        return 0

    lax.fori_loop(1, NPAIRS, pair_body, 0)
    pltpu.make_async_copy(sbuf0, acc.at[sridx0], ssem0).wait()
    pltpu.make_async_copy(sbuf1, acc.at[sridx1], ssem1).wait()
    plsc.subcore_barrier()
    pltpu.sync_copy(acc.at[pl.ds(tid * ROWS_PT, ROWS_PT)],
                    out_hbm.at[cid, pl.ds(tid * ROWS_PT, ROWS_PT)])


_spmm_call = pl.kernel(
    _spmm_body,
    out_type=jax.ShapeDtypeStruct((NUM_SC, NPAD, HALF), jnp.float32),
    mesh=plsc.VectorSubcoreMesh(core_axis_name="c", subcore_axis_name="s"),
    scratch_types=[
        pltpu.MemorySpace.VMEM_SHARED((NPAD, HALF), jnp.float32),
        pltpu.VMEM((BLK,), jnp.int32),
        pltpu.VMEM((BLK,), jnp.int32),
        pltpu.VMEM((BLK,), jnp.float32),
    ] + [pltpu.VMEM((CHUNK, HALF), jnp.float32) for _ in range(4)]
    + [pltpu.VMEM((CHUNK,), jnp.int32) for _ in range(4)]
    + [pltpu.SemaphoreType.DMA for _ in range(4)],
)


def _spmm(x, rows, cols, ev, zeros):
    return _spmm_call(x, rows, cols, ev, zeros)


def kernel(edge_index, edge_values, embed, W1, b1, Wh, bh, Wt, bt):
    pad = E_PAD - E
    spread = jnp.arange(pad, dtype=jnp.int32) % N
    rows = jnp.concatenate([edge_index[0], spread])
    cols = jnp.concatenate([edge_index[1], spread])
    ev = jnp.concatenate([edge_values, jnp.zeros((pad,), jnp.float32)])
    zeros = jnp.zeros((ROWS_PT, HALF), jnp.float32)
    b1r = b1.reshape(NUM_SC, 1, HALF)
    bhr = bh.reshape(NUM_SC, 1, HALF)
    btr = bt.reshape(1, C)

    x1 = _mm1(embed, W1, b1r)
    y1 = _spmm(x1, rows, cols, ev, zeros)
    x2 = _mm2(y1, Wh, bhr)
    y2 = _spmm(x2, rows, cols, ev, zeros)
    return _mm3(y2, Wt, btr)


# speedup vs baseline: 2.6423x; 1.0517x over previous
import functools

import jax
import jax.numpy as jnp
from jax import lax
from jax.experimental import pallas as pl
from jax.experimental.pallas import tpu as pltpu
from jax.experimental.pallas import tpu_sc as plsc

N = 10000
E = 320000
F_IN = 128
H = 256
C = 64

NUM_SC = 2
NUM_TILES = 16
HALF = H // 2
E_PAD = 327680
EPT = E_PAD // NUM_TILES
BLK = 2560
N_BLKS = EPT // BLK
CHUNK = 80
CPB = BLK // CHUNK
N_CHUNKS = EPT // CHUNK
NPAIRS = N_CHUNKS // 2
NPAD = 10240
ROWS_PT = NPAD // NUM_TILES
MT = 1000


def _mm1_body(x_ref, w_ref, b_ref, o_ref):
    res = lax.dot_general(
        x_ref[...], w_ref[...], (((1,), (1,)), ((), ())),
        preferred_element_type=jnp.float32, precision=lax.Precision.HIGHEST)
    o_ref[...] = res[None] + b_ref[...]


def _mm1(x, w, b2):
    return pl.pallas_call(
        _mm1_body,
        grid=(NUM_SC, N // MT),
        in_specs=[
            pl.BlockSpec((MT, F_IN), lambda h, m: (m, 0)),
            pl.BlockSpec((HALF, F_IN), lambda h, m: (h, 0)),
            pl.BlockSpec((1, 1, HALF), lambda h, m: (h, 0, 0)),
        ],
        out_specs=pl.BlockSpec((1, MT, HALF), lambda h, m: (h, m, 0)),
        out_shape=jax.ShapeDtypeStruct((NUM_SC, N, HALF), jnp.float32),
    )(x, w, b2)


def _mm2_body(y_ref, w_ref, b_ref, o_ref):
    yv = jnp.maximum(
        jnp.concatenate([y_ref[0], y_ref[1]], axis=1), 0.0)
    part = lax.dot_general(
        yv, w_ref[...], (((1,), (1,)), ((), ())),
        preferred_element_type=jnp.float32, precision=lax.Precision.HIGHEST)
    o_ref[...] = part[None] + b_ref[...]


def _mm2(y, w, b2):
    return pl.pallas_call(
        _mm2_body,
        grid=(NUM_SC, N // MT),
        in_specs=[
            pl.BlockSpec((NUM_SC, MT, HALF), lambda h, m: (0, m, 0)),
            pl.BlockSpec((HALF, H), lambda h, m: (h, 0)),
            pl.BlockSpec((1, 1, HALF), lambda h, m: (h, 0, 0)),
        ],
        out_specs=pl.BlockSpec((1, MT, HALF), lambda h, m: (h, m, 0)),
        out_shape=jax.ShapeDtypeStruct((NUM_SC, N, HALF), jnp.float32),
    )(y, w, b2)


def _mm3_body(y_ref, w_ref, b_ref, o_ref):
    yv = jnp.maximum(
        jnp.concatenate([y_ref[0], y_ref[1]], axis=1), 0.0)
    part = lax.dot_general(
        yv, w_ref[...], (((1,), (1,)), ((), ())),
        preferred_element_type=jnp.float32, precision=lax.Precision.HIGHEST)
    z = jnp.maximum(part + b_ref[...], 0.0)
    m = jnp.max(z, axis=-1, keepdims=True)
    s = z - m
    lse = jnp.log(jnp.sum(jnp.exp(s), axis=-1, keepdims=True))
    o_ref[...] = s - lse


def _mm3(y, w, b2):
    return pl.pallas_call(
        _mm3_body,
        grid=(N // MT,),
        in_specs=[
            pl.BlockSpec((NUM_SC, MT, HALF), lambda m: (0, m, 0)),
            pl.BlockSpec((C, H), lambda m: (0, 0)),
            pl.BlockSpec((1, C), lambda m: (0, 0)),
        ],
        out_specs=pl.BlockSpec((MT, C), lambda m: (m, 0)),
        out_shape=jax.ShapeDtypeStruct((N, C), jnp.float32),
    )(y, w, b2)


def _spmm_body(x_hbm, rows_hbm, cols_hbm, ev_hbm, zeros_hbm, out_hbm,
               acc, rows_v, cols_v, ev_v,
               gbuf0, gbuf1, sbuf0, sbuf1,
               sridx0, sridx1, cidx0, cidx1,
               gsem0, gsem1, ssem0, ssem1):
    tid = lax.axis_index("s")
    cid = lax.axis_index("c")
    pltpu.sync_copy(zeros_hbm, acc.at[pl.ds(tid * ROWS_PT, ROWS_PT)])
    plsc.subcore_barrier()

    x_h = x_hbm.at[cid]
    dn = lax.GatherDimensionNumbers(
        offset_dims=(), collapsed_slice_dims=(0,), start_index_map=(0,))

    def stage_block(bi):
        base = tid * EPT + bi * BLK
        pltpu.sync_copy(rows_hbm.at[pl.ds(base, BLK)], rows_v)
        pltpu.sync_copy(cols_hbm.at[pl.ds(base, BLK)], cols_v)
        pltpu.sync_copy(ev_hbm.at[pl.ds(base, BLK)], ev_v)

    def fire_gather(c, cidx, gbuf, gsem):
        off = (c % CPB) * CHUNK
        for j in range(CHUNK // 16):
            cidx[pl.ds(j * 16, 16)] = cols_v[pl.ds(off + j * 16, 16)]
        pltpu.async_copy(x_h.at[cidx], gbuf, gsem)

    def scale(c, gbuf, sbuf):
        off = (c % CPB) * CHUNK

        @plsc.parallel_loop(0, CHUNK // 16, unroll=2)
        def _(g):
            evg = ev_v[pl.ds(off + g * 16, 16)]
            row0 = g * 16
            for l in range(16):
                evb = lax.gather(
                    evg, jnp.full((16, 1), l, jnp.int32), dn,
                    slice_sizes=(1,),
                    mode=lax.GatherScatterMode.PROMISE_IN_BOUNDS)
                for j in range(HALF // 16):
                    sl = pl.ds(j * 16, 16)
                    sbuf[row0 + l, sl] = gbuf[row0 + l, sl] * evb

    def fire_scatter(c, sridx, sbuf, ssem):
        off = (c % CPB) * CHUNK
        for j in range(CHUNK // 16):
            sridx[pl.ds(j * 16, 16)] = rows_v[pl.ds(off + j * 16, 16)]
        pltpu.async_copy(sbuf, acc.at[sridx], ssem, add=True)

    stage_block(0)
    fire_gather(0, cidx0, gbuf0, gsem0)
    fire_gather(1, cidx1, gbuf1, gsem1)
    pltpu.make_async_copy(x_h.at[cidx0], gbuf0, gsem0).wait()
    scale(0, gbuf0, sbuf0)
    fire_scatter(0, sridx0, sbuf0, ssem0)
    fire_gather(2, cidx0, gbuf0, gsem0)
    pltpu.make_async_copy(x_h.at[cidx1], gbuf1, gsem1).wait()
    scale(1, gbuf1, sbuf1)
    fire_scatter(1, sridx1, sbuf1, ssem1)
    fire_gather(3, cidx1, gbuf1, gsem1)

    def pair_body(p, _):
        c0 = 2 * p
        c1 = c0 + 1
        nxt = p < NPAIRS - 1
        boundary = (c0 + 2) % CPB == 0
        pltpu.make_async_copy(x_h.at[cidx0], gbuf0, gsem0).wait()
        pltpu.make_async_copy(sbuf0, acc.at[sridx0], ssem0).wait()
        scale(c0, gbuf0, sbuf0)
        fire_scatter(c0, sridx0, sbuf0, ssem0)

        @pl.when(jnp.logical_and(nxt, jnp.logical_not(boundary)))
        def _():
            fire_gather(c0 + 2, cidx0, gbuf0, gsem0)

        pltpu.make_async_copy(x_h.at[cidx1], gbuf1, gsem1).wait()
        pltpu.make_async_copy(sbuf1, acc.at[sridx1], ssem1).wait()
        scale(c1, gbuf1, sbuf1)
        fire_scatter(c1, sridx1, sbuf1, ssem1)

        @pl.when(jnp.logical_and(nxt, boundary))
        def _():
            stage_block((c0 + 2) // CPB)
            fire_gather(c0 + 2, cidx0, gbuf0, gsem0)

        @pl.when(nxt)
        def _():
            fire_gather(c1 + 2, cidx1, gbuf1, gsem1)

        return 0

    lax.fori_loop(1, NPAIRS, pair_body, 0)
    pltpu.make_async_copy(sbuf0, acc.at[sridx0], ssem0).wait()
    pltpu.make_async_copy(sbuf1, acc.at[sridx1], ssem1).wait()
    plsc.subcore_barrier()
    pltpu.sync_copy(acc.at[pl.ds(tid * ROWS_PT, ROWS_PT)],
                    out_hbm.at[cid, pl.ds(tid * ROWS_PT, ROWS_PT)])


_spmm_call = pl.kernel(
    _spmm_body,
    out_type=jax.ShapeDtypeStruct((NUM_SC, NPAD, HALF), jnp.float32),
    mesh=plsc.VectorSubcoreMesh(core_axis_name="c", subcore_axis_name="s"),
    scratch_types=[
        pltpu.MemorySpace.VMEM_SHARED((NPAD, HALF), jnp.float32),
        pltpu.VMEM((BLK,), jnp.int32),
        pltpu.VMEM((BLK,), jnp.int32),
        pltpu.VMEM((BLK,), jnp.float32),
    ] + [pltpu.VMEM((CHUNK, HALF), jnp.float32) for _ in range(4)]
    + [pltpu.VMEM((CHUNK,), jnp.int32) for _ in range(4)]
    + [pltpu.SemaphoreType.DMA for _ in range(4)],
)


def _spmm(x, rows, cols, ev, zeros):
    return _spmm_call(x, rows, cols, ev, zeros)


def kernel(edge_index, edge_values, embed, W1, b1, Wh, bh, Wt, bt):
    pad = E_PAD - E
    spread = jnp.arange(pad, dtype=jnp.int32) % N
    rows = jnp.concatenate([edge_index[0], spread])
    cols = jnp.concatenate([edge_index[1], spread])
    ev = jnp.concatenate([edge_values, jnp.zeros((pad,), jnp.float32)])
    zeros = jnp.zeros((ROWS_PT, HALF), jnp.float32)
    b1r = b1.reshape(NUM_SC, 1, HALF)
    bhr = bh.reshape(NUM_SC, 1, HALF)
    btr = bt.reshape(1, C)

    x1 = _mm1(embed, W1, b1r)
    y1 = _spmm(x1, rows, cols, ev, zeros)
    x2 = _mm2(y1, Wh, bhr)
    y2 = _spmm(x2, rows, cols, ev, zeros)
    return _mm3(y2, Wt, btr)
